# baseline (speedup 1.0000x reference)
import jax
import jax.numpy as jnp
from jax.experimental import pallas as pl
import jax.experimental.pallas.tpu as pltpu

E = 16
TOPK = 2
H = 1024
FF = 2048
T = 128
HH = H // 2
FH = FF // 2


def _moe_kernel(hs_ref, rw_ref, gu_ref, down_ref, sg_ref, su_ref, sd_ref,
                out_ref, scores_out_ref, scores_scr, actsh_scr,
                gup_scr, act_scr):
    s = pl.program_id(0)

    @pl.when(s == 0)
    def _():
        hs = hs_ref[...]
        logits = jax.lax.dot_general(
            hs, rw_ref[...], (((1,), (1,)), ((), ())),
            preferred_element_type=jnp.float32)
        iota_e = jax.lax.broadcasted_iota(jnp.int32, (T, E), 1)
        m1 = jnp.max(logits, axis=1, keepdims=True)
        pos1 = jnp.min(jnp.where(logits == m1, iota_e, E), axis=1,
                       keepdims=True)
        oh1 = iota_e == pos1
        masked = jnp.where(oh1, -jnp.inf, logits)
        m2 = jnp.max(masked, axis=1, keepdims=True)
        pos2 = jnp.min(jnp.where(masked == m2, iota_e, E), axis=1,
                       keepdims=True)
        oh2 = iota_e == pos2
        sel = jnp.logical_or(oh1, oh2)
        scores_te = jnp.where(sel, jax.nn.sigmoid(logits), 0.0)
        scores_scr[...] = scores_te
        scores_out_ref[...] = scores_te.T
        hsb = hs.astype(jnp.bfloat16)
        gsh = jax.lax.dot_general(hsb, sg_ref[...].astype(jnp.bfloat16),
                                  (((1,), (1,)), ((), ())),
                                  preferred_element_type=jnp.float32)
        ush = jax.lax.dot_general(hsb, su_ref[...].astype(jnp.bfloat16),
                                  (((1,), (1,)), ((), ())),
                                  preferred_element_type=jnp.float32)
        actsh_scr[...] = jax.nn.silu(gsh) * ush

    @pl.when(s == 1)
    def _():
        out_ref[...] = jax.lax.dot_general(
            actsh_scr[...].astype(jnp.bfloat16),
            sd_ref[...].astype(jnp.bfloat16),
            (((1,), (1,)), ((), ())),
            preferred_element_type=jnp.float32)

    e = jnp.maximum(s - 2, 0) // 4
    sub = jnp.maximum(s - 2, 0) % 4

    @pl.when(jnp.logical_and(s >= 2, sub <= 1))
    def _():
        iota_e = jax.lax.broadcasted_iota(jnp.int32, (T, E), 1)
        sc = jnp.sum(jnp.where(iota_e == e, scores_scr[...], 0.0),
                     axis=1, keepdims=True)
        x = (hs_ref[...] * sc).astype(jnp.bfloat16)
        gu = gu_ref[0, 0].astype(jnp.bfloat16)

        @pl.when(sub == 0)
        def _():
            gup_scr[...] = jax.lax.dot_general(
                x[:, :HH], gu, (((1,), (0,)), ((), ())),
                preferred_element_type=jnp.float32)

        @pl.when(sub == 1)
        def _():
            gup = gup_scr[...] + jax.lax.dot_general(
                x[:, HH:], gu, (((1,), (0,)), ((), ())),
                preferred_element_type=jnp.float32)
            act_scr[...] = (jax.nn.silu(gup[:, :FF])
                            * gup[:, FF:]).astype(jnp.bfloat16)

    @pl.when(sub == 2)
    def _():
        out_ref[...] += jax.lax.dot_general(
            act_scr[...][:, :FH], down_ref[0, 0].astype(jnp.bfloat16),
            (((1,), (0,)), ((), ())),
            preferred_element_type=jnp.float32)

    @pl.when(jnp.logical_and(s >= 2, sub == 3))
    def _():
        out_ref[...] += jax.lax.dot_general(
            act_scr[...][:, FH:], down_ref[0, 0].astype(jnp.bfloat16),
            (((1,), (0,)), ((), ())),
            preferred_element_type=jnp.float32)


def _sub(g):
    return jnp.maximum(g - 2, 0) % 4


def _e_idx(g):
    return jnp.maximum(g - 2, 0) // 4


@jax.jit
def kernel(hidden_states, router_w, gate_up_proj, down_proj,
           shared_gate_w, shared_up_w, shared_down_w):
    hs = hidden_states.reshape(-1, H)
    gu4 = gate_up_proj.reshape(E, 2, HH, 2 * FF)
    dn4 = down_proj.reshape(E, 2, FH, H)

    out, router_scores = pl.pallas_call(
        _moe_kernel,
        grid=(2 + 4 * E,),
        in_specs=[
            pl.BlockSpec((T, H), lambda g: (0, 0)),
            pl.BlockSpec((E, H), lambda g: (0, 0)),
            pl.BlockSpec((1, 1, HH, 2 * FF),
                         lambda g: (_e_idx(g), jnp.minimum(_sub(g), 1),
                                    0, 0)),
            pl.BlockSpec((1, 1, FH, H),
                         lambda g: (_e_idx(g),
                                    jnp.clip(_sub(g) - 2, 0, 1), 0, 0)),
            pl.BlockSpec((FF, H), lambda g: (0, 0)),
            pl.BlockSpec((FF, H), lambda g: (0, 0)),
            pl.BlockSpec((H, FF), lambda g: (0, 0)),
        ],
        out_specs=[
            pl.BlockSpec((T, H), lambda g: (0, 0)),
            pl.BlockSpec((E, T), lambda g: (0, 0)),
        ],
        out_shape=[
            jax.ShapeDtypeStruct((T, H), jnp.float32),
            jax.ShapeDtypeStruct((E, T), jnp.float32),
        ],
        scratch_shapes=[
            pltpu.VMEM((T, E), jnp.float32),
            pltpu.VMEM((T, FF), jnp.float32),
            pltpu.VMEM((T, 2 * FF), jnp.float32),
            pltpu.VMEM((T, FF), jnp.bfloat16),
        ],
        compiler_params=pltpu.CompilerParams(
            dimension_semantics=("arbitrary",),
            vmem_limit_bytes=60 * 1024 * 1024,
        ),
    )(hs, router_w, gu4, dn4, shared_gate_w, shared_up_w, shared_down_w)

    return (out, router_scores)


# speedup vs baseline: 1.2136x; 1.2136x over previous
import jax
import jax.numpy as jnp
from jax.experimental import pallas as pl
import jax.experimental.pallas.tpu as pltpu

E = 16
TOPK = 2
H = 1024
FF = 2048
T = 128


def _router_shared_kernel(hs_ref, rw_ref, sg_ref, su_ref, sd_ref,
                          scores_out_ref, scores_te_ref, shared_ref):
    hs = hs_ref[...]
    logits = jax.lax.dot_general(
        hs, rw_ref[...], (((1,), (1,)), ((), ())),
        preferred_element_type=jnp.float32)
    iota_e = jax.lax.broadcasted_iota(jnp.int32, (T, E), 1)
    m1 = jnp.max(logits, axis=1, keepdims=True)
    pos1 = jnp.min(jnp.where(logits == m1, iota_e, E), axis=1, keepdims=True)
    oh1 = iota_e == pos1
    masked = jnp.where(oh1, -jnp.inf, logits)
    m2 = jnp.max(masked, axis=1, keepdims=True)
    pos2 = jnp.min(jnp.where(masked == m2, iota_e, E), axis=1, keepdims=True)
    oh2 = iota_e == pos2
    sel = jnp.logical_or(oh1, oh2)
    scores_te = jnp.where(sel, jax.nn.sigmoid(logits), 0.0)
    scores_te_ref[...] = scores_te
    scores_out_ref[...] = scores_te.T
    hsb = hs.astype(jnp.bfloat16)
    gsh = jax.lax.dot_general(hsb, sg_ref[...].astype(jnp.bfloat16),
                              (((1,), (1,)), ((), ())),
                              preferred_element_type=jnp.float32)
    ush = jax.lax.dot_general(hsb, su_ref[...].astype(jnp.bfloat16),
                              (((1,), (1,)), ((), ())),
                              preferred_element_type=jnp.float32)
    act = (jax.nn.silu(gsh) * ush).astype(jnp.bfloat16)
    shared_ref[...] = jax.lax.dot_general(
        act, sd_ref[...].astype(jnp.bfloat16), (((1,), (1,)), ((), ())),
        preferred_element_type=jnp.float32)


def _expert_kernel(scores_te_ref, hs_ref, shared_ref, gu_ref, down_ref,
                   out_ref):
    e = pl.program_id(0)

    @pl.when(e == 0)
    def _():
        out_ref[...] = shared_ref[...]

    iota_e = jax.lax.broadcasted_iota(jnp.int32, (T, E), 1)
    sc = jnp.sum(jnp.where(iota_e == e, scores_te_ref[...], 0.0),
                 axis=1, keepdims=True)
    x = (hs_ref[...] * sc).astype(jnp.bfloat16)
    gu = gu_ref[0].astype(jnp.bfloat16)
    gmat = jax.lax.dot_general(x, gu[:, :FF], (((1,), (0,)), ((), ())),
                               preferred_element_type=jnp.float32)
    umat = jax.lax.dot_general(x, gu[:, FF:], (((1,), (0,)), ((), ())),
                               preferred_element_type=jnp.float32)
    act = (jax.nn.silu(gmat) * umat).astype(jnp.bfloat16)
    out_ref[...] += jax.lax.dot_general(
        act, down_ref[0].astype(jnp.bfloat16), (((1,), (0,)), ((), ())),
        preferred_element_type=jnp.float32)


@jax.jit
def kernel(hidden_states, router_w, gate_up_proj, down_proj,
           shared_gate_w, shared_up_w, shared_down_w):
    hs = hidden_states.reshape(-1, H)

    router_scores, scores_te, shared_out = pl.pallas_call(
        _router_shared_kernel,
        out_shape=[
            jax.ShapeDtypeStruct((E, T), jnp.float32),
            jax.ShapeDtypeStruct((T, E), jnp.float32),
            jax.ShapeDtypeStruct((T, H), jnp.float32),
        ],
    )(hs, router_w, shared_gate_w, shared_up_w, shared_down_w)

    out = pl.pallas_call(
        _expert_kernel,
        grid=(E,),
        in_specs=[
            pl.BlockSpec((T, E), lambda e: (0, 0)),
            pl.BlockSpec((T, H), lambda e: (0, 0)),
            pl.BlockSpec((T, H), lambda e: (0, 0)),
            pl.BlockSpec((1, H, 2 * FF), lambda e: (e, 0, 0)),
            pl.BlockSpec((1, FF, H), lambda e: (e, 0, 0)),
        ],
        out_specs=pl.BlockSpec((T, H), lambda e: (0, 0)),
        out_shape=jax.ShapeDtypeStruct((T, H), jnp.float32),
        compiler_params=pltpu.CompilerParams(
            dimension_semantics=("arbitrary",),
            vmem_limit_bytes=60 * 1024 * 1024,
        ),
    )(scores_te, hs, shared_out, gate_up_proj, down_proj)

    return (out, router_scores)


# speedup vs baseline: 1.2419x; 1.0233x over previous
import jax
import jax.numpy as jnp
from jax.experimental import pallas as pl
import jax.experimental.pallas.tpu as pltpu

E = 16
TOPK = 2
H = 1024
FF = 2048
T = 128

FB = 1024
FFB = FF // FB


def _moe_kernel(hs_ref, rw_ref, gate_ref, up_ref, down_ref,
                sg_ref, su_ref, sd_ref,
                out_ref, scores_out_ref, scores_scr, act_scr):
    g = pl.program_id(0)

    @pl.when(g == 0)
    def _():
        hs = hs_ref[...]
        logits = jax.lax.dot_general(
            hs, rw_ref[...], (((1,), (1,)), ((), ())),
            preferred_element_type=jnp.float32)
        iota_e = jax.lax.broadcasted_iota(jnp.int32, (T, E), 1)
        m1 = jnp.max(logits, axis=1, keepdims=True)
        pos1 = jnp.min(jnp.where(logits == m1, iota_e, E), axis=1,
                       keepdims=True)
        oh1 = iota_e == pos1
        masked = jnp.where(oh1, -jnp.inf, logits)
        m2 = jnp.max(masked, axis=1, keepdims=True)
        pos2 = jnp.min(jnp.where(masked == m2, iota_e, E), axis=1,
                       keepdims=True)
        oh2 = iota_e == pos2
        sel = jnp.logical_or(oh1, oh2)
        scores_te = jnp.where(sel, jax.nn.sigmoid(logits), 0.0)
        scores_scr[...] = scores_te
        scores_out_ref[...] = scores_te.T
        hsb = hs.astype(jnp.bfloat16)
        gsh = jax.lax.dot_general(hsb, sg_ref[...].astype(jnp.bfloat16),
                                  (((1,), (1,)), ((), ())),
                                  preferred_element_type=jnp.float32)
        ush = jax.lax.dot_general(hsb, su_ref[...].astype(jnp.bfloat16),
                                  (((1,), (1,)), ((), ())),
                                  preferred_element_type=jnp.float32)
        act_scr[...] = jax.nn.silu(gsh) * ush

    @pl.when(g == 1)
    def _():
        out_ref[...] = jax.lax.dot_general(
            act_scr[...].astype(jnp.bfloat16),
            sd_ref[...].astype(jnp.bfloat16),
            (((1,), (1,)), ((), ())),
            preferred_element_type=jnp.float32)

    @pl.when(g >= 2)
    def _():
        e = (g - 2) // FFB
        f = (g - 2) % FFB
        iota_e = jax.lax.broadcasted_iota(jnp.int32, (T, E), 1)
        sc = jnp.sum(jnp.where(iota_e == e, scores_scr[...], 0.0),
                     axis=1, keepdims=True)
        x = (hs_ref[...] * sc).astype(jnp.bfloat16)
        gmat = jax.lax.dot_general(x, gate_ref[0].astype(jnp.bfloat16),
                                   (((1,), (0,)), ((), ())),
                                   preferred_element_type=jnp.float32)
        umat = jax.lax.dot_general(x, up_ref[0].astype(jnp.bfloat16),
                                   (((1,), (0,)), ((), ())),
                                   preferred_element_type=jnp.float32)
        act = (jax.nn.silu(gmat) * umat).astype(jnp.bfloat16)

        @pl.when(f == 0)
        def _():
            out_ref[...] += jax.lax.dot_general(
                act, down_ref[0, :FB].astype(jnp.bfloat16),
                (((1,), (0,)), ((), ())),
                preferred_element_type=jnp.float32)

        @pl.when(f == 1)
        def _():
            out_ref[...] += jax.lax.dot_general(
                act, down_ref[0, FB:].astype(jnp.bfloat16),
                (((1,), (0,)), ((), ())),
                preferred_element_type=jnp.float32)


def _e_idx(g):
    return jnp.maximum(g - 2, 0) // FFB


def _f_idx(g):
    return jnp.maximum(g - 2, 0) % FFB


@jax.jit
def kernel(hidden_states, router_w, gate_up_proj, down_proj,
           shared_gate_w, shared_up_w, shared_down_w):
    hs = hidden_states.reshape(-1, H)

    out, router_scores = pl.pallas_call(
        _moe_kernel,
        grid=(2 + E * FFB,),
        in_specs=[
            pl.BlockSpec((T, H), lambda g: (0, 0)),
            pl.BlockSpec((E, H), lambda g: (0, 0)),
            pl.BlockSpec((1, H, FB), lambda g: (_e_idx(g), 0, _f_idx(g))),
            pl.BlockSpec((1, H, FB),
                         lambda g: (_e_idx(g), 0, _f_idx(g) + FFB)),
            pl.BlockSpec((1, FF, H), lambda g: (_e_idx(g), 0, 0)),
            pl.BlockSpec((FF, H), lambda g: (0, 0)),
            pl.BlockSpec((FF, H), lambda g: (0, 0)),
            pl.BlockSpec((H, FF), lambda g: (0, 0)),
        ],
        out_specs=[
            pl.BlockSpec((T, H), lambda g: (0, 0)),
            pl.BlockSpec((E, T), lambda g: (0, 0)),
        ],
        out_shape=[
            jax.ShapeDtypeStruct((T, H), jnp.float32),
            jax.ShapeDtypeStruct((E, T), jnp.float32),
        ],
        scratch_shapes=[
            pltpu.VMEM((T, E), jnp.float32),
            pltpu.VMEM((T, FF), jnp.float32),
        ],
        compiler_params=pltpu.CompilerParams(
            dimension_semantics=("arbitrary",),
            vmem_limit_bytes=63 * 1024 * 1024,
        ),
    )(hs, router_w, gate_up_proj, gate_up_proj, down_proj,
      shared_gate_w, shared_up_w, shared_down_w)

    return (out, router_scores)


# speedup vs baseline: 1.2644x; 1.0181x over previous
import jax
import jax.numpy as jnp
from jax.experimental import pallas as pl
import jax.experimental.pallas.tpu as pltpu

E = 16
TOPK = 2
H = 1024
FF = 2048
T = 128

FB = 1024
FFB = FF // FB


def _moe_kernel(hs_ref, rw_ref, gate_ref, up_ref, down_ref,
                sg_ref, su_ref, sd_ref,
                out_ref, scores_out_ref, scores_scr, act_scr):
    g = pl.program_id(0)

    @pl.when(g == 0)
    def _():
        hs = hs_ref[...]
        logits = jax.lax.dot_general(
            hs, rw_ref[...], (((1,), (1,)), ((), ())),
            preferred_element_type=jnp.float32)
        iota_e = jax.lax.broadcasted_iota(jnp.int32, (T, E), 1)
        m1 = jnp.max(logits, axis=1, keepdims=True)
        pos1 = jnp.min(jnp.where(logits == m1, iota_e, E), axis=1,
                       keepdims=True)
        oh1 = iota_e == pos1
        masked = jnp.where(oh1, -jnp.inf, logits)
        m2 = jnp.max(masked, axis=1, keepdims=True)
        pos2 = jnp.min(jnp.where(masked == m2, iota_e, E), axis=1,
                       keepdims=True)
        oh2 = iota_e == pos2
        sel = jnp.logical_or(oh1, oh2)
        scores_te = jnp.where(sel, jax.nn.sigmoid(logits), 0.0)
        scores_scr[...] = scores_te
        scores_out_ref[...] = scores_te.T
        hsb = hs.astype(jnp.bfloat16)
        gsh = jax.lax.dot_general(hsb, sg_ref[...].astype(jnp.bfloat16),
                                  (((1,), (1,)), ((), ())),
                                  preferred_element_type=jnp.float32)
        ush = jax.lax.dot_general(hsb, su_ref[...].astype(jnp.bfloat16),
                                  (((1,), (1,)), ((), ())),
                                  preferred_element_type=jnp.float32)
        act_scr[...] = jax.nn.silu(gsh) * ush

    @pl.when(g == 1)
    def _():
        out_ref[...] = jax.lax.dot_general(
            act_scr[...].astype(jnp.bfloat16),
            sd_ref[...].astype(jnp.bfloat16),
            (((1,), (1,)), ((), ())),
            preferred_element_type=jnp.float32)

    @pl.when(g >= 2)
    def _():
        e = (g - 2) // FFB
        iota_e = jax.lax.broadcasted_iota(jnp.int32, (T, E), 1)
        sc = jnp.sum(jnp.where(iota_e == e, scores_scr[...], 0.0),
                     axis=1, keepdims=True)
        x = (hs_ref[...] * sc).astype(jnp.bfloat16)
        gmat = jax.lax.dot_general(x, gate_ref[0].astype(jnp.bfloat16),
                                   (((1,), (0,)), ((), ())),
                                   preferred_element_type=jnp.float32)
        umat = jax.lax.dot_general(x, up_ref[0].astype(jnp.bfloat16),
                                   (((1,), (0,)), ((), ())),
                                   preferred_element_type=jnp.float32)
        act = (jax.nn.silu(gmat) * umat).astype(jnp.bfloat16)
        out_ref[...] += jax.lax.dot_general(
            act, down_ref[0].astype(jnp.bfloat16),
            (((1,), (0,)), ((), ())),
            preferred_element_type=jnp.float32)


def _e_idx(g):
    return jnp.maximum(g - 2, 0) // FFB


def _f_idx(g):
    return jnp.maximum(g - 2, 0) % FFB


@jax.jit
def kernel(hidden_states, router_w, gate_up_proj, down_proj,
           shared_gate_w, shared_up_w, shared_down_w):
    hs = hidden_states.reshape(-1, H)

    out, router_scores = pl.pallas_call(
        _moe_kernel,
        grid=(2 + E * FFB,),
        in_specs=[
            pl.BlockSpec((T, H), lambda g: (0, 0)),
            pl.BlockSpec((E, H), lambda g: (0, 0)),
            pl.BlockSpec((1, H, FB), lambda g: (_e_idx(g), 0, _f_idx(g))),
            pl.BlockSpec((1, H, FB),
                         lambda g: (_e_idx(g), 0, _f_idx(g) + FFB)),
            pl.BlockSpec((1, FB, H), lambda g: (_e_idx(g), _f_idx(g), 0)),
            pl.BlockSpec((FF, H), lambda g: (0, 0)),
            pl.BlockSpec((FF, H), lambda g: (0, 0)),
            pl.BlockSpec((H, FF), lambda g: (0, 0)),
        ],
        out_specs=[
            pl.BlockSpec((T, H), lambda g: (0, 0)),
            pl.BlockSpec((E, T), lambda g: (0, 0)),
        ],
        out_shape=[
            jax.ShapeDtypeStruct((T, H), jnp.float32),
            jax.ShapeDtypeStruct((E, T), jnp.float32),
        ],
        scratch_shapes=[
            pltpu.VMEM((T, E), jnp.float32),
            pltpu.VMEM((T, FF), jnp.float32),
        ],
        compiler_params=pltpu.CompilerParams(
            dimension_semantics=("arbitrary",),
            vmem_limit_bytes=60 * 1024 * 1024,
        ),
    )(hs, router_w, gate_up_proj, gate_up_proj, down_proj,
      shared_gate_w, shared_up_w, shared_down_w)

    return (out, router_scores)


# speedup vs baseline: 1.2652x; 1.0007x over previous
import jax
import jax.numpy as jnp
from jax.experimental import pallas as pl
import jax.experimental.pallas.tpu as pltpu

E = 16
TOPK = 2
H = 1024
FF = 2048
T = 128

FB = 1024
FFB = FF // FB
NEXP = E * FFB
GRID = 1 + NEXP + FFB


def _moe_kernel(hs_ref, rw_ref, gate_ref, up_ref, down_ref,
                sg_ref, su_ref, sd_ref,
                out_ref, scores_out_ref, scores_scr):
    g = pl.program_id(0)

    @pl.when(g == 0)
    def _():
        hs = hs_ref[...]
        logits = jax.lax.dot_general(
            hs, rw_ref[...], (((1,), (1,)), ((), ())),
            preferred_element_type=jnp.float32)
        iota_e = jax.lax.broadcasted_iota(jnp.int32, (T, E), 1)
        m1 = jnp.max(logits, axis=1, keepdims=True)
        pos1 = jnp.min(jnp.where(logits == m1, iota_e, E), axis=1,
                       keepdims=True)
        oh1 = iota_e == pos1
        masked = jnp.where(oh1, -jnp.inf, logits)
        m2 = jnp.max(masked, axis=1, keepdims=True)
        pos2 = jnp.min(jnp.where(masked == m2, iota_e, E), axis=1,
                       keepdims=True)
        oh2 = iota_e == pos2
        sel = jnp.logical_or(oh1, oh2)
        scores_te = jnp.where(sel, jax.nn.sigmoid(logits), 0.0)
        scores_scr[...] = scores_te
        scores_out_ref[...] = scores_te.T

    @pl.when(jnp.logical_and(g >= 1, g <= NEXP))
    def _():
        e = (g - 1) // FFB
        iota_e = jax.lax.broadcasted_iota(jnp.int32, (T, E), 1)
        sc = jnp.sum(jnp.where(iota_e == e, scores_scr[...], 0.0),
                     axis=1, keepdims=True)
        x = (hs_ref[...] * sc).astype(jnp.bfloat16)
        gmat = jax.lax.dot_general(x, gate_ref[0].astype(jnp.bfloat16),
                                   (((1,), (0,)), ((), ())),
                                   preferred_element_type=jnp.float32)
        umat = jax.lax.dot_general(x, up_ref[0].astype(jnp.bfloat16),
                                   (((1,), (0,)), ((), ())),
                                   preferred_element_type=jnp.float32)
        act = (jax.nn.silu(gmat) * umat).astype(jnp.bfloat16)
        contrib = jax.lax.dot_general(
            act, down_ref[0].astype(jnp.bfloat16),
            (((1,), (0,)), ((), ())),
            preferred_element_type=jnp.float32)

        @pl.when(g == 1)
        def _():
            out_ref[...] = contrib

        @pl.when(g > 1)
        def _():
            out_ref[...] += contrib

    @pl.when(g > NEXP)
    def _():
        xb = hs_ref[...].astype(jnp.bfloat16)
        gs = jax.lax.dot_general(xb, sg_ref[...].astype(jnp.bfloat16),
                                 (((1,), (1,)), ((), ())),
                                 preferred_element_type=jnp.float32)
        us = jax.lax.dot_general(xb, su_ref[...].astype(jnp.bfloat16),
                                 (((1,), (1,)), ((), ())),
                                 preferred_element_type=jnp.float32)
        act = (jax.nn.silu(gs) * us).astype(jnp.bfloat16)
        out_ref[...] += jax.lax.dot_general(
            act, sd_ref[...].astype(jnp.bfloat16),
            (((1,), (1,)), ((), ())),
            preferred_element_type=jnp.float32)


def _s_idx(g):
    return jnp.clip(g - 1, 0, NEXP - 1)


def _e_idx(g):
    return _s_idx(g) // FFB


def _f_idx(g):
    return _s_idx(g) % FFB


def _j_idx(g):
    return jnp.clip(g - 1 - NEXP, 0, FFB - 1)


@jax.jit
def kernel(hidden_states, router_w, gate_up_proj, down_proj,
           shared_gate_w, shared_up_w, shared_down_w):
    hs = hidden_states.reshape(-1, H)

    out, router_scores = pl.pallas_call(
        _moe_kernel,
        grid=(GRID,),
        in_specs=[
            pl.BlockSpec((T, H), lambda g: (0, 0)),
            pl.BlockSpec((E, H), lambda g: (0, 0)),
            pl.BlockSpec((1, H, FB), lambda g: (_e_idx(g), 0, _f_idx(g))),
            pl.BlockSpec((1, H, FB),
                         lambda g: (_e_idx(g), 0, _f_idx(g) + FFB)),
            pl.BlockSpec((1, FB, H), lambda g: (_e_idx(g), _f_idx(g), 0)),
            pl.BlockSpec((FB, H), lambda g: (_j_idx(g), 0)),
            pl.BlockSpec((FB, H), lambda g: (_j_idx(g), 0)),
            pl.BlockSpec((H, FB), lambda g: (0, _j_idx(g))),
        ],
        out_specs=[
            pl.BlockSpec((T, H), lambda g: (0, 0)),
            pl.BlockSpec((E, T), lambda g: (0, 0)),
        ],
        out_shape=[
            jax.ShapeDtypeStruct((T, H), jnp.float32),
            jax.ShapeDtypeStruct((E, T), jnp.float32),
        ],
        scratch_shapes=[
            pltpu.VMEM((T, E), jnp.float32),
        ],
        compiler_params=pltpu.CompilerParams(
            dimension_semantics=("arbitrary",),
            vmem_limit_bytes=100 * 1024 * 1024,
        ),
    )(hs, router_w, gate_up_proj, gate_up_proj, down_proj,
      shared_gate_w, shared_up_w, shared_down_w)

    return (out, router_scores)
